# baseline (speedup 1.0000x reference)
import functools

import jax
import jax.numpy as jnp
from jax import lax
from jax.experimental import pallas as pl
from jax.experimental.pallas import tpu as pltpu
from jax.experimental.pallas import tpu_sc as plsc

VOCAB = 100000
EMBD = 128
CTX = 10
HID = 512
NIDX = 2 * CTX

K_T = 16
NK = HID // K_T
NBUF = 4

SC_CW = 2048
SC_KT = 8
SC_NB = 6
SC_NCH = HID // SC_KT


def _sc_stream_probe(W2):
    mesh = plsc.VectorSubcoreMesh(core_axis_name="c", subcore_axis_name="s")

    @functools.partial(
        pl.kernel,
        mesh=mesh,
        out_type=jax.ShapeDtypeStruct((32, 16), jnp.float32),
        scratch_types=[
            pltpu.VMEM((SC_NB, SC_KT, SC_CW), jnp.float32),
            pltpu.SemaphoreType.DMA((SC_NB,)),
        ],
    )
    def k(w2_hbm, out_hbm, bufs, sems):
        wid = lax.axis_index("s") * 2 + lax.axis_index("c")
        c0 = wid * SC_CW

        def mk(i):
            return pltpu.make_async_copy(
                w2_hbm.at[pl.ds(i * SC_KT, SC_KT), pl.ds(c0, SC_CW)],
                bufs.at[i % SC_NB],
                sems.at[i % SC_NB],
            )

        for s in range(SC_NB - 1):
            mk(s).start()
        for i in range(SC_NCH):
            mk(i).wait()
            nxt = i + SC_NB - 1
            if nxt < SC_NCH:
                mk(nxt).start()
        pltpu.sync_copy(bufs.at[0, 0, pl.ds(0, 16)], out_hbm.at[wid])

    return k(W2)


def _hid_body(e_ref, w1_ref, b1_ref, o_ref):
    o_ref[...] = jnp.maximum(
        jnp.dot(e_ref[...], w1_ref[...], preferred_element_type=jnp.float32)
        + b1_ref[...],
        0.0,
    )


def _out_body(hid_ref, b2_ref, w2_hbm, o_ref, bufs, sems):
    o_ref[...] = b2_ref[...] + hid_ref[0, 0] + bufs[0, 0:1, 0:1]


def kernel(inputs, table, W1, b1, W2, b2):
    idx = inputs.astype(jnp.int32)
    emb = jnp.take(table, idx, axis=0)
    emb_flat = emb.reshape(1, NIDX * EMBD)

    probe = _sc_stream_probe(W2)

    hid = pl.pallas_call(
        _hid_body,
        out_shape=jax.ShapeDtypeStruct((1, HID), jnp.float32),
    )(emb_flat, W1, b1.reshape(1, HID))

    log_probs = pl.pallas_call(
        _out_body,
        in_specs=[
            pl.BlockSpec((1, HID), lambda: (0, 0)),
            pl.BlockSpec((1, VOCAB), lambda: (0, 0)),
            pl.BlockSpec(memory_space=pl.ANY),
        ],
        out_specs=pl.BlockSpec((1, VOCAB), lambda: (0, 0)),
        out_shape=jax.ShapeDtypeStruct((1, VOCAB), jnp.float32),
        scratch_shapes=[
            pltpu.VMEM((NBUF, K_T, VOCAB), jnp.float32),
            pltpu.SemaphoreType.DMA((NBUF,)),
        ],
    )(hid, b2.reshape(1, VOCAB), W2)

    return log_probs + jnp.sum(probe) * 1e-38


# speedup vs baseline: 1.0751x; 1.0751x over previous
import functools

import jax
import jax.numpy as jnp
from jax import lax
from jax.experimental import pallas as pl
from jax.experimental.pallas import tpu as pltpu
from jax.experimental.pallas import tpu_sc as plsc

VOCAB = 100000
EMBD = 128
CTX = 10
HID = 512
NIDX = 2 * CTX

NQ = 8
NBUF = 2


def _sc_gather(table, idx):
    mesh = plsc.VectorSubcoreMesh(core_axis_name="c", subcore_axis_name="s")

    @functools.partial(
        pl.kernel,
        mesh=mesh,
        out_type=jax.ShapeDtypeStruct((NIDX, EMBD), jnp.float32),
        scratch_types=[
            pltpu.VMEM((NIDX,), jnp.int32),
            pltpu.VMEM((NIDX, EMBD), jnp.float32),
            pltpu.SemaphoreType.DMA,
        ],
    )
    def gather_k(table_hbm, idx_hbm, out_hbm, idx_v, rows_v, sem):
        wid = lax.axis_index("s") * 2 + lax.axis_index("c")

        @pl.when(wid == 0)
        def _():
            pltpu.sync_copy(idx_hbm, idx_v)
            pltpu.async_copy(table_hbm.at[idx_v], rows_v, sem).wait()
            pltpu.sync_copy(rows_v, out_hbm)

    return gather_k(table, idx)


def _hid_body(e_ref, w1_ref, b1_ref, o_ref):
    o_ref[...] = jnp.maximum(
        jnp.dot(e_ref[...], w1_ref[...], preferred_element_type=jnp.float32)
        + b1_ref[...],
        0.0,
    )


def _out_body(hid_ref, b2_ref, w2_hbm, o_ref, bufs, sems):
    def mk(q):
        return pltpu.make_async_copy(
            w2_hbm.at[:, q, :, :], bufs.at[q % NBUF], sems.at[q % NBUF]
        )

    for s in range(NBUF - 1):
        mk(s).start()
    for q in range(NQ):
        mk(q).wait()
        w = bufs[q % NBUF].reshape(64, VOCAB)
        t = jnp.dot(hid_ref[q], w, preferred_element_type=jnp.float32)
        if q == 0:
            o_ref[...] = t + b2_ref[...]
        else:
            o_ref[...] = o_ref[...] + t
        nxt = q + NBUF - 1
        if nxt < NQ:
            mk(nxt).start()

    full = o_ref[...]
    m = jnp.max(full)
    s = jnp.sum(jnp.exp(full - m))
    o_ref[...] = full - (m + jnp.log(s))


def kernel(inputs, table, W1, b1, W2, b2):
    idx = inputs.astype(jnp.int32)
    emb = _sc_gather(table, idx)
    emb_flat = emb.reshape(1, NIDX * EMBD)

    hid = pl.pallas_call(
        _hid_body,
        out_shape=jax.ShapeDtypeStruct((1, HID), jnp.float32),
    )(emb_flat, W1, b1.reshape(1, HID))

    hid_p = jnp.transpose(hid.reshape(8, 8, 8), (1, 0, 2)).reshape(NQ, 1, 64)

    log_probs = pl.pallas_call(
        _out_body,
        in_specs=[
            pl.BlockSpec((NQ, 1, 64), lambda: (0, 0, 0)),
            pl.BlockSpec((1, VOCAB), lambda: (0, 0)),
            pl.BlockSpec(memory_space=pl.ANY),
        ],
        out_specs=pl.BlockSpec((1, VOCAB), lambda: (0, 0)),
        out_shape=jax.ShapeDtypeStruct((1, VOCAB), jnp.float32),
        scratch_shapes=[
            pltpu.VMEM((NBUF, NQ, 8, VOCAB), jnp.float32),
            pltpu.SemaphoreType.DMA((NBUF,)),
        ],
    )(hid_p, b2.reshape(1, VOCAB), W2.reshape(NQ, 8, 8, VOCAB))

    return log_probs


# speedup vs baseline: 1.1294x; 1.0505x over previous
import functools

import jax
import jax.numpy as jnp
from jax import lax
from jax.experimental import pallas as pl
from jax.experimental.pallas import tpu as pltpu
from jax.experimental.pallas import tpu_sc as plsc

VOCAB = 100000
EMBD = 128
CTX = 10
HID = 512
NIDX = 2 * CTX

NQ = 16
NST = 4
NBUF = 3


def _sc_gather(table, idx):
    mesh = plsc.VectorSubcoreMesh(core_axis_name="c", subcore_axis_name="s")

    @functools.partial(
        pl.kernel,
        mesh=mesh,
        out_type=jax.ShapeDtypeStruct((NIDX, EMBD), jnp.float32),
        scratch_types=[
            pltpu.VMEM((NIDX,), jnp.int32),
            pltpu.VMEM((NIDX, EMBD), jnp.float32),
            pltpu.SemaphoreType.DMA,
        ],
    )
    def gather_k(table_hbm, idx_hbm, out_hbm, idx_v, rows_v, sem):
        wid = lax.axis_index("s") * 2 + lax.axis_index("c")

        @pl.when(wid == 0)
        def _():
            pltpu.sync_copy(idx_hbm, idx_v)
            pltpu.async_copy(table_hbm.at[idx_v], rows_v, sem).wait()
            pltpu.sync_copy(rows_v, out_hbm)

    return gather_k(table, idx)


def _hid_body(e_ref, w1_ref, b1_ref, o_ref):
    o_ref[...] = jnp.maximum(
        jnp.dot(e_ref[...], w1_ref[...], preferred_element_type=jnp.float32)
        + b1_ref[...],
        0.0,
    )


def _out_body(hid_ref, b2_ref, w2_hbm, o_ref, bufs, sems):
    def mk(q):
        return pltpu.make_async_copy(
            w2_hbm.at[:, q, :, :], bufs.at[q % NBUF], sems.at[q % NBUF]
        )

    for s in range(NBUF - 1):
        mk(s).start()
    for q in range(NQ):
        mk(q).wait()
        w = bufs[q % NBUF].reshape(NST * 8, VOCAB)
        t = jnp.dot(hid_ref[q], w, preferred_element_type=jnp.float32)
        if q == 0:
            o_ref[...] = t + b2_ref[...]
        else:
            o_ref[...] = o_ref[...] + t
        nxt = q + NBUF - 1
        if nxt < NQ:
            mk(nxt).start()

    full = o_ref[...]
    m = jnp.max(full)
    s = jnp.sum(jnp.exp(full - m))
    o_ref[...] = full - (m + jnp.log(s))


def kernel(inputs, table, W1, b1, W2, b2):
    idx = inputs.astype(jnp.int32)
    emb = _sc_gather(table, idx)
    emb_flat = emb.reshape(1, NIDX * EMBD)

    hid = pl.pallas_call(
        _hid_body,
        out_shape=jax.ShapeDtypeStruct((1, HID), jnp.float32),
    )(emb_flat, W1, b1.reshape(1, HID))

    hid_p = jnp.transpose(hid.reshape(NST, NQ, 8), (1, 0, 2)).reshape(NQ, 1, NST * 8)

    log_probs = pl.pallas_call(
        _out_body,
        in_specs=[
            pl.BlockSpec((NQ, 1, NST * 8), lambda: (0, 0, 0)),
            pl.BlockSpec((1, VOCAB), lambda: (0, 0)),
            pl.BlockSpec(memory_space=pl.ANY),
        ],
        out_specs=pl.BlockSpec((1, VOCAB), lambda: (0, 0)),
        out_shape=jax.ShapeDtypeStruct((1, VOCAB), jnp.float32),
        scratch_shapes=[
            pltpu.VMEM((NBUF, NST, 8, VOCAB), jnp.float32),
            pltpu.SemaphoreType.DMA((NBUF,)),
        ],
    )(hid_p, b2.reshape(1, VOCAB), W2.reshape(NST, NQ, 8, VOCAB))

    return log_probs
